# baseline (speedup 1.0000x reference)
import jax
import jax.numpy as jnp
from jax.experimental import pallas as pl
from jax.experimental.pallas import tpu as pltpu

_N_CHUNKS = 8


def _copy_body(x_hbm, o_hbm, sems):
    n_rows = x_hbm.shape[0]
    chunk = (n_rows + _N_CHUNKS - 1) // _N_CHUNKS
    copies = []
    for i in range(_N_CHUNKS):
        lo = i * chunk
        hi = min(lo + chunk, n_rows)
        cp = pltpu.make_async_copy(
            x_hbm.at[pl.ds(lo, hi - lo), :],
            o_hbm.at[pl.ds(lo, hi - lo), :],
            sems.at[i],
        )
        cp.start()
        copies.append(cp)
    for cp in copies:
        cp.wait()


def kernel(code_embedding):
    return pl.pallas_call(
        _copy_body,
        in_specs=[pl.BlockSpec(memory_space=pltpu.HBM)],
        out_specs=pl.BlockSpec(memory_space=pltpu.HBM),
        scratch_shapes=[pltpu.SemaphoreType.DMA((_N_CHUNKS,))],
        out_shape=jax.ShapeDtypeStruct(code_embedding.shape, code_embedding.dtype),
    )(code_embedding)


# speedup vs baseline: 14.6722x; 14.6722x over previous
import functools

import jax
import jax.numpy as jnp
from jax import lax
from jax.experimental import pallas as pl
from jax.experimental.pallas import tpu as pltpu
from jax.experimental.pallas import tpu_sc as plsc

_N_ROWS = 1000001
_DIM = 32
_FLAT = _N_ROWS * _DIM
_N_WORKERS = 32
_PER_WORKER = 1000000
_TAIL = _FLAT - _N_WORKERS * _PER_WORKER
_NBUF = 4
_CHUNK = 25000
_NCHUNKS = _PER_WORKER // _CHUNK


def _copy_kernel(x_hbm, out_hbm, b0, b1, b2, b3, in_sems, out_sems):
    bufs = (b0, b1, b2, b3)
    wid = lax.axis_index("s") * 2 + lax.axis_index("c")
    base = wid * _PER_WORKER

    def in_copy(c):
        return pltpu.make_async_copy(
            x_hbm.at[pl.ds(base + c * _CHUNK, _CHUNK)],
            bufs[c % _NBUF],
            in_sems.at[c % _NBUF],
        )

    def out_copy(c):
        return pltpu.make_async_copy(
            bufs[c % _NBUF],
            out_hbm.at[pl.ds(base + c * _CHUNK, _CHUNK)],
            out_sems.at[c % _NBUF],
        )

    for b in range(_NBUF):
        in_copy(b).start()
    for c in range(_NCHUNKS):
        in_copy(c).wait()
        out_copy(c).start()
        if c + _NBUF < _NCHUNKS:
            out_copy(c).wait()
            in_copy(c + _NBUF).start()
    for c in range(_NCHUNKS - _NBUF, _NCHUNKS):
        out_copy(c).wait()

    @pl.when(wid == _N_WORKERS - 1)
    def _():
        tail_base = _N_WORKERS * _PER_WORKER
        tin = pltpu.make_async_copy(
            x_hbm.at[pl.ds(tail_base, _TAIL)],
            b0.at[pl.ds(0, _TAIL)],
            in_sems.at[0],
        )
        tin.start()
        tin.wait()
        tout = pltpu.make_async_copy(
            b0.at[pl.ds(0, _TAIL)],
            out_hbm.at[pl.ds(tail_base, _TAIL)],
            out_sems.at[0],
        )
        tout.start()
        tout.wait()


def kernel(code_embedding):
    mesh = plsc.VectorSubcoreMesh(core_axis_name="c", subcore_axis_name="s")
    k = functools.partial(
        pl.kernel,
        mesh=mesh,
        out_type=jax.ShapeDtypeStruct((_FLAT,), jnp.float32),
        scratch_types=(
            [pltpu.VMEM((_CHUNK,), jnp.float32) for _ in range(_NBUF)]
            + [pltpu.SemaphoreType.DMA((_NBUF,)), pltpu.SemaphoreType.DMA((_NBUF,))]
        ),
    )(_copy_kernel)
    flat = code_embedding.reshape(_FLAT)
    return k(flat).reshape(_N_ROWS, _DIM)
